# baseline (speedup 1.0000x reference)
import jax
import jax.numpy as jnp
from jax.experimental import pallas as pl
from jax.experimental.pallas import tpu as pltpu

_CHUNK = 2048


def _copy_body(w_hbm, o_hbm, buf, *sems):
    n = w_hbm.shape[0] // _CHUNK
    nb = o_hbm.shape[0]

    def in_cp(k):
        return pltpu.make_async_copy(
            w_hbm.at[pl.ds(k * _CHUNK, _CHUNK), :],
            buf.at[pl.ds(k * _CHUNK, _CHUNK), :],
            sems[k],
        )

    def out_cp(b, k):
        return pltpu.make_async_copy(
            buf.at[pl.ds(k * _CHUNK, _CHUNK), :],
            o_hbm.at[b, pl.ds(k * _CHUNK, _CHUNK), :],
            sems[-1],
        )

    for k in range(n):
        in_cp(k).start()
    for b in range(nb):
        for k in range(n):
            if b == 0:
                in_cp(k).wait()
            out_cp(b, k).start()
    for b in range(nb):
        for k in range(n):
            out_cp(b, k).wait()


def kernel(input_ids, embed_weight):
    batch, seq_len = input_ids.shape
    _, embed_dim = embed_weight.shape
    table = embed_weight[:seq_len]
    out = pl.pallas_call(
        _copy_body,
        in_specs=[pl.BlockSpec(memory_space=pl.ANY)],
        out_specs=pl.BlockSpec(memory_space=pl.ANY),
        out_shape=jax.ShapeDtypeStruct((batch, seq_len, embed_dim), table.dtype),
        scratch_shapes=(
            [pltpu.VMEM((seq_len, embed_dim), table.dtype)]
            + [pltpu.SemaphoreType.DMA for _ in range(seq_len // _CHUNK + 1)]
        ),
    )(table)
    return out


# speedup vs baseline: 1.0622x; 1.0622x over previous
import jax
import jax.numpy as jnp
from jax.experimental import pallas as pl
from jax.experimental.pallas import tpu as pltpu

_CHUNK = 2048


def _copy_body(w_hbm, o_hbm, buf, *sems):
    n = w_hbm.shape[0] // _CHUNK
    nb = o_hbm.shape[0]

    def in_cp(k):
        return pltpu.make_async_copy(
            w_hbm.at[pl.ds(k * _CHUNK, _CHUNK), :],
            buf.at[pl.ds(k * _CHUNK, _CHUNK), :],
            sems[k],
        )

    def out_cp(b, k):
        return pltpu.make_async_copy(
            buf.at[pl.ds(k * _CHUNK, _CHUNK), :],
            o_hbm.at[b, pl.ds(k * _CHUNK, _CHUNK), :],
            sems[-1],
        )

    for k in range(n):
        in_cp(k).start()
    for k in range(n):
        in_cp(k).wait()
        for b in range(nb):
            out_cp(b, k).start()
    for b in range(nb):
        for k in range(n):
            out_cp(b, k).wait()


def kernel(input_ids, embed_weight):
    batch, seq_len = input_ids.shape
    _, embed_dim = embed_weight.shape
    table = embed_weight[:seq_len]
    out = pl.pallas_call(
        _copy_body,
        in_specs=[pl.BlockSpec(memory_space=pl.ANY)],
        out_specs=pl.BlockSpec(memory_space=pl.ANY),
        out_shape=jax.ShapeDtypeStruct((batch, seq_len, embed_dim), table.dtype),
        scratch_shapes=(
            [pltpu.VMEM((seq_len, embed_dim), table.dtype)]
            + [pltpu.SemaphoreType.DMA for _ in range(seq_len // _CHUNK + 1)]
        ),
    )(table)
    return out


# speedup vs baseline: 1.0639x; 1.0016x over previous
import jax
import jax.numpy as jnp
from jax.experimental import pallas as pl
from jax.experimental.pallas import tpu as pltpu

_CHUNK = 2048


_NBUF = 4
_LAG = 3


def _copy_body(w_hbm, o_hbm, *refs):
    bufs = refs[:_NBUF]
    in_sems = refs[_NBUF : 2 * _NBUF]
    out_sems = refs[2 * _NBUF : 3 * _NBUF]
    n = w_hbm.shape[0] // _CHUNK
    nb = o_hbm.shape[0]

    def in_cp(k):
        return pltpu.make_async_copy(
            w_hbm.at[pl.ds(k * _CHUNK, _CHUNK), :], bufs[k % _NBUF], in_sems[k % _NBUF]
        )

    def out_cp(k, b):
        return pltpu.make_async_copy(
            bufs[k % _NBUF],
            o_hbm.at[b, pl.ds(k * _CHUNK, _CHUNK), :],
            out_sems[k % _NBUF],
        )

    for j in range(min(_NBUF - _LAG, n)):
        in_cp(j).start()
    for k in range(n):
        if k >= _LAG:
            for b in range(nb):
                out_cp(k - _LAG, b).wait()
        if k + _NBUF - _LAG < n:
            in_cp(k + _NBUF - _LAG).start()
        in_cp(k).wait()
        for b in range(nb):
            out_cp(k, b).start()
    for k in range(max(0, n - _LAG), n):
        for b in range(nb):
            out_cp(k, b).wait()


def kernel(input_ids, embed_weight):
    batch, seq_len = input_ids.shape
    _, embed_dim = embed_weight.shape
    table = embed_weight[:seq_len]
    out = pl.pallas_call(
        _copy_body,
        in_specs=[pl.BlockSpec(memory_space=pl.ANY)],
        out_specs=pl.BlockSpec(memory_space=pl.ANY),
        out_shape=jax.ShapeDtypeStruct((batch, seq_len, embed_dim), table.dtype),
        scratch_shapes=(
            [pltpu.VMEM((_CHUNK, embed_dim), table.dtype) for _ in range(_NBUF)]
            + [pltpu.SemaphoreType.DMA for _ in range(2 * _NBUF)]
        ),
    )(table)
    return out
